# baseline (speedup 1.0000x reference)
import functools

import jax
import jax.numpy as jnp
from jax import lax
from jax.experimental import pallas as pl
from jax.experimental.pallas import tpu as pltpu
from jax.experimental.pallas import tpu_sc as plsc

A = 8192
P = 4096
K = 16
NPART = 16
L = 16
NC = 2
NS = 16
NW = NC * NS
APW = A // NW
BIG = 1e30


def _sc_topk(ax, ay, av, ap, px, py, lo_t, hi_t):
    mesh = plsc.VectorSubcoreMesh(
        core_axis_name="c", subcore_axis_name="s",
        num_cores=NC, num_subcores=NS)

    @functools.partial(
        pl.kernel,
        out_type=jax.ShapeDtypeStruct((2, A * K), jnp.int32),
        mesh=mesh,
        compiler_params=pltpu.CompilerParams(needs_layout_passes=False),
        scratch_types=[
            pltpu.VMEM((P + 2 * L,), jnp.float32),
            pltpu.VMEM((P + 2 * L,), jnp.float32),
            pltpu.VMEM((APW,), jnp.float32),
            pltpu.VMEM((APW,), jnp.float32),
            pltpu.VMEM((APW,), jnp.int32),
            pltpu.VMEM((APW,), jnp.int32),
            pltpu.VMEM((NPART,), jnp.int32),
            pltpu.VMEM((NPART,), jnp.int32),
            pltpu.VMEM((APW * K,), jnp.int32),
            pltpu.VMEM((APW * K,), jnp.int32),
        ],
    )
    def k(ax_h, ay_h, av_h, ap_h, px_h, py_h, lo_h, hi_h,
          out_h,
          px_v, py_v, ax_v, ay_v, av_v, ap_v, lo_v, hi_v, o0_v, o1_v):
        wid = lax.axis_index("s") * NC + lax.axis_index("c")
        base = wid * APW
        pltpu.sync_copy(px_h, px_v)
        pltpu.sync_copy(py_h, py_v)
        pltpu.sync_copy(lo_h, lo_v)
        pltpu.sync_copy(hi_h, hi_v)
        pltpu.sync_copy(ax_h.at[pl.ds(base, APW)], ax_v)
        pltpu.sync_copy(ay_h.at[pl.ds(base, APW)], ay_v)
        pltpu.sync_copy(av_h.at[pl.ds(base, APW)], av_v)
        pltpu.sync_copy(ap_h.at[pl.ds(base, APW)], ap_v)

        iota = lax.iota(jnp.int32, L)
        lo_r = lo_v[...]
        hi_r = hi_v[...]

        def _splat(vec, lane_v):
            return jnp.take_along_axis(vec, lane_v, axis=0,
                                       mode="promise_in_bounds")

        def one_chunk(j0, bd, bi, ax_b, ay_b, hi_b):
            jv = jnp.full((L,), j0, jnp.int32) + iota
            dx = px_v[pl.ds(j0, L)] - ax_b
            dy = py_v[pl.ds(j0, L)] - ay_b
            d2 = dx * dx + dy * dy
            d2 = jnp.where(jv < hi_b, d2, BIG)
            sd, si = plsc.sort_key_val(d2, jv)
            rd = lax.rev(sd, (0,))
            ri = lax.rev(si, (0,))
            keep = bd <= rd
            nd = jnp.where(keep, bd, rd)
            ni = jnp.where(keep, bi, ri)
            md, mi = plsc.sort_key_val(nd, ni)
            return md, mi

        def epilogue(a, bi, lo_b, hi_b):
            cnt_b = hi_b - lo_b
            fm = iota - cnt_b
            fi = jnp.where(fm < lo_b, fm, hi_b + (fm - lo_b))
            outi = jnp.where(iota < cnt_b, bi, fi)
            o0_v[pl.ds(a * K, K)] = outi
            o1_v[pl.ds(a * K, K)] = jnp.full((L,), base + a, jnp.int32)

        bd_init = jnp.full((L,), BIG, jnp.float32)
        bi_init = jnp.zeros((L,), jnp.int32)
        clamp = jnp.int32(P)

        def pair_body(g, _):
            a0 = 2 * g
            grp = a0 & jnp.int32(-L)
            l0 = a0 - grp
            axg = ax_v[pl.ds(grp, L)]
            ayg = ay_v[pl.ds(grp, L)]
            ptg = ap_v[pl.ds(grp, L)]
            avg = av_v[pl.ds(grp, L)]
            okg = avg > 0
            log = jnp.where(okg, _splat(lo_r, ptg), 0)
            hig = jnp.where(okg, _splat(hi_r, ptg), 0)
            nchg = lax.shift_right_logical(hig - log + (L - 1), 4)
            packg = jnp.left_shift(nchg, 16) | log
            lane0 = jnp.full((L,), l0, jnp.int32)
            lane1 = lane0 + 1
            ax0 = _splat(axg, lane0)
            ay0 = _splat(ayg, lane0)
            ax1 = _splat(axg, lane1)
            ay1 = _splat(ayg, lane1)
            lo0 = _splat(log, lane0)
            hi0 = _splat(hig, lane0)
            lo1 = _splat(log, lane1)
            hi1 = _splat(hig, lane1)
            pk0 = _splat(packg, lane0)[0]
            pk1 = _splat(packg, lane1)[0]
            c00 = pk0 & jnp.int32(0xFFFF)
            nch0 = lax.shift_right_logical(pk0, 16)
            c01 = pk1 & jnp.int32(0xFFFF)
            nch1 = lax.shift_right_logical(pk1, 16)
            n = jnp.maximum(nch0, nch1)

            def chunk(t, carry):
                bd0, bi0, bd1, bi1 = carry
                toff = t * L
                j00 = jnp.minimum(c00 + toff, clamp)
                j01 = jnp.minimum(c01 + toff, clamp)
                bd0, bi0 = one_chunk(j00, bd0, bi0, ax0, ay0, hi0)
                bd1, bi1 = one_chunk(j01, bd1, bi1, ax1, ay1, hi1)
                return (bd0, bi0, bd1, bi1)

            bd0, bi0, bd1, bi1 = plsc.parallel_loop(
                0, n, carry=(bd_init, bi_init, bd_init, bi_init),
                unroll=2)(chunk)
            epilogue(a0, bi0, lo0, hi0)
            epilogue(a0 + 1, bi1, lo1, hi1)
            return 0

        lax.fori_loop(0, APW // 2, pair_body, 0)
        pltpu.sync_copy(o0_v, out_h.at[0, pl.ds(base * K, APW * K)])
        pltpu.sync_copy(o1_v, out_h.at[1, pl.ds(base * K, APW * K)])

    return k(ax, ay, av, ap, px, py, lo_t, hi_t)


def kernel(agent_position, agent_valid, agent_partition,
           polyline_start_position, polyline_partition):
    ax = agent_position[:, 0].astype(jnp.float32)
    ay = agent_position[:, 1].astype(jnp.float32)
    av = agent_valid.astype(jnp.int32)
    ap = agent_partition.astype(jnp.int32)
    px = jnp.pad(polyline_start_position[:, 0].astype(jnp.float32), (0, 2 * L))
    py = jnp.pad(polyline_start_position[:, 1].astype(jnp.float32), (0, 2 * L))
    ids = jnp.arange(NPART, dtype=polyline_partition.dtype)
    lo_t = jnp.searchsorted(polyline_partition, ids, side="left").astype(jnp.int32)
    hi_t = jnp.searchsorted(polyline_partition, ids, side="right").astype(jnp.int32)
    return _sc_topk(ax, ay, av, ap, px, py, lo_t, hi_t)


# speedup vs baseline: 1.1422x; 1.1422x over previous
import functools

import jax
import jax.numpy as jnp
from jax import lax
from jax.experimental import pallas as pl
from jax.experimental.pallas import tpu as pltpu
from jax.experimental.pallas import tpu_sc as plsc

A = 8192
P = 4096
K = 16
NPART = 16
L = 16
NC = 2
NS = 16
NW = NC * NS
APW = A // NW
BIG = 1e30


def _sc_topk(ax, ay, av, ap, px, py, lo_t, hi_t):
    mesh = plsc.VectorSubcoreMesh(
        core_axis_name="c", subcore_axis_name="s",
        num_cores=NC, num_subcores=NS)

    @functools.partial(
        pl.kernel,
        out_type=jax.ShapeDtypeStruct((2, A * K), jnp.int32),
        mesh=mesh,
        compiler_params=pltpu.CompilerParams(needs_layout_passes=False),
        scratch_types=[
            pltpu.VMEM((P + 2 * L,), jnp.float32),
            pltpu.VMEM((P + 2 * L,), jnp.float32),
            pltpu.VMEM((APW,), jnp.float32),
            pltpu.VMEM((APW,), jnp.float32),
            pltpu.VMEM((APW,), jnp.int32),
            pltpu.VMEM((APW,), jnp.int32),
            pltpu.VMEM((NPART,), jnp.int32),
            pltpu.VMEM((NPART,), jnp.int32),
            pltpu.VMEM((APW * K,), jnp.int32),
            pltpu.VMEM((APW * K,), jnp.int32),
        ],
    )
    def k(ax_h, ay_h, av_h, ap_h, px_h, py_h, lo_h, hi_h,
          out_h,
          px_v, py_v, ax_v, ay_v, av_v, ap_v, lo_v, hi_v, o0_v, o1_v):
        wid = lax.axis_index("s") * NC + lax.axis_index("c")
        base = wid * APW
        pltpu.sync_copy(px_h, px_v)
        pltpu.sync_copy(py_h, py_v)
        pltpu.sync_copy(lo_h, lo_v)
        pltpu.sync_copy(hi_h, hi_v)
        pltpu.sync_copy(ax_h.at[pl.ds(base, APW)], ax_v)
        pltpu.sync_copy(ay_h.at[pl.ds(base, APW)], ay_v)
        pltpu.sync_copy(av_h.at[pl.ds(base, APW)], av_v)
        pltpu.sync_copy(ap_h.at[pl.ds(base, APW)], ap_v)

        iota = lax.iota(jnp.int32, L)
        lo_r = lo_v[...]
        hi_r = hi_v[...]

        def _splat(vec, lane_v):
            return jnp.take_along_axis(vec, lane_v, axis=0,
                                       mode="promise_in_bounds")

        def one_chunk(j0, bd, bi, ax_b, ay_b, hi_b):
            jv = jnp.full((L,), j0, jnp.int32) + iota
            dx = px_v[pl.ds(j0, L)] - ax_b
            dy = py_v[pl.ds(j0, L)] - ay_b
            d2 = dx * dx + dy * dy
            d2 = jnp.where(jv < hi_b, d2, BIG)
            sd, si = plsc.sort_key_val(d2, jv)
            rd = lax.rev(sd, (0,))
            ri = lax.rev(si, (0,))
            keep = bd <= rd
            nd = jnp.where(keep, bd, rd)
            ni = jnp.where(keep, bi, ri)
            md, mi = plsc.sort_key_val(nd, ni)
            return md, mi

        def epilogue(a, bi, lo_b, hi_b):
            cnt_b = hi_b - lo_b
            fm = iota - cnt_b
            fi = jnp.where(fm < lo_b, fm, hi_b + (fm - lo_b))
            outi = jnp.where(iota < cnt_b, bi, fi)
            o0_v[pl.ds(a * K, K)] = outi
            o1_v[pl.ds(a * K, K)] = jnp.full((L,), base + a, jnp.int32)

        bd_init = jnp.full((L,), BIG, jnp.float32)
        bi_init = jnp.zeros((L,), jnp.int32)
        clamp = jnp.int32(P)

        NA = 4

        def quad_body(g, _):
            a0 = NA * g
            grp = a0 & jnp.int32(-L)
            l0 = a0 - grp
            axg = ax_v[pl.ds(grp, L)]
            ayg = ay_v[pl.ds(grp, L)]
            ptg = ap_v[pl.ds(grp, L)]
            avg = av_v[pl.ds(grp, L)]
            okg = avg > 0
            log = jnp.where(okg, _splat(lo_r, ptg), 0)
            hig = jnp.where(okg, _splat(hi_r, ptg), 0)
            nchg = lax.shift_right_logical(hig - log + (L - 1), 4)
            packg = jnp.left_shift(nchg, 16) | log
            lane0 = jnp.full((L,), l0, jnp.int32)
            ax_a, ay_a, lo_a, hi_a, c0_a, nch_a = [], [], [], [], [], []
            for q in range(NA):
                lane = lane0 + q
                ax_a.append(_splat(axg, lane))
                ay_a.append(_splat(ayg, lane))
                lo_a.append(_splat(log, lane))
                hi_a.append(_splat(hig, lane))
                pk = _splat(packg, lane)[0]
                c0_a.append(pk & jnp.int32(0xFFFF))
                nch_a.append(lax.shift_right_logical(pk, 16))
            n = nch_a[0]
            for q in range(1, NA):
                n = jnp.maximum(n, nch_a[q])

            def chunk(t, carry):
                out = []
                toff = t * L
                for q in range(NA):
                    bd, bi = carry[2 * q], carry[2 * q + 1]
                    j0 = jnp.minimum(c0_a[q] + toff, clamp)
                    bd, bi = one_chunk(j0, bd, bi, ax_a[q], ay_a[q], hi_a[q])
                    out.extend((bd, bi))
                return tuple(out)

            res = lax.fori_loop(0, n, chunk, (bd_init, bi_init) * NA)
            for q in range(NA):
                epilogue(a0 + q, res[2 * q + 1], lo_a[q], hi_a[q])
            return 0

        lax.fori_loop(0, APW // NA, quad_body, 0)
        pltpu.sync_copy(o0_v, out_h.at[0, pl.ds(base * K, APW * K)])
        pltpu.sync_copy(o1_v, out_h.at[1, pl.ds(base * K, APW * K)])

    return k(ax, ay, av, ap, px, py, lo_t, hi_t)


def kernel(agent_position, agent_valid, agent_partition,
           polyline_start_position, polyline_partition):
    ax = agent_position[:, 0].astype(jnp.float32)
    ay = agent_position[:, 1].astype(jnp.float32)
    av = agent_valid.astype(jnp.int32)
    ap = agent_partition.astype(jnp.int32)
    px = jnp.pad(polyline_start_position[:, 0].astype(jnp.float32), (0, 2 * L))
    py = jnp.pad(polyline_start_position[:, 1].astype(jnp.float32), (0, 2 * L))
    ids = jnp.arange(NPART, dtype=polyline_partition.dtype)
    lo_t = jnp.searchsorted(polyline_partition, ids, side="left").astype(jnp.int32)
    hi_t = jnp.searchsorted(polyline_partition, ids, side="right").astype(jnp.int32)
    return _sc_topk(ax, ay, av, ap, px, py, lo_t, hi_t)
